# baseline (speedup 1.0000x reference)
import functools

import jax
import jax.numpy as jnp
from jax import lax
from jax.experimental import pallas as pl
from jax.experimental.pallas import tpu as pltpu
from jax.experimental.pallas import tpu_sc as plsc

_NC = 2
_NS = 16
_L = 16
_R = 16


@functools.cache
def _make_sc_loss(b, t):
    nw = _NC * _NS
    bw = b // nw
    nchunk = bw // _R
    nfull = t // _L
    assert nchunk % 2 == 0 and nchunk >= 4 and t % _L != 0
    mesh = plsc.VectorSubcoreMesh(core_axis_name="c", subcore_axis_name="s")

    @functools.partial(
        pl.kernel,
        mesh=mesh,
        out_type=jax.ShapeDtypeStruct((nw, _L), jnp.float32),
        scratch_types=[
            pltpu.VMEM((bw,), jnp.int32),
            pltpu.VMEM((2, _R, t), jnp.float32),
            pltpu.VMEM((_L,), jnp.float32),
            pltpu.SemaphoreType.DMA,
            pltpu.SemaphoreType.DMA,
        ],
    )
    def sc_loss(x_hbm, c_hbm, out_hbm, c_v, buf_v, acc_v, sem0, sem1):
        wid = lax.axis_index("s") * _NC + lax.axis_index("c")
        base = wid * bw
        pltpu.sync_copy(c_hbm.at[pl.ds(base, bw)], c_v)

        lanes = lax.iota(jnp.int32, _L)
        tailmask = jnp.where(lanes >= _L - (t % _L), 1.0, 0.0).astype(jnp.float32)
        sems = (sem0, sem1)

        pltpu.async_copy(x_hbm.at[pl.ds(base, _R)], buf_v.at[0], sem0)
        pltpu.async_copy(x_hbm.at[pl.ds(base + _R, _R)], buf_v.at[1], sem1)

        def consume(par, k, accs2, accg):
            pltpu.make_async_copy(
                x_hbm.at[pl.ds(0, _R)], buf_v.at[par], sems[par]
            ).wait()
            acc = accs2
            g = accg
            cols = c_v[pl.ds(pl.multiple_of(k * _R, _L), _L)]
            for r in range(_R):
                for o in range(nfull):
                    x = buf_v[par, r, pl.ds(o * _L, _L)]
                    acc = acc + x * x
                xt = buf_v[par, r, pl.ds(t - _L, _L)] * tailmask
                acc = acc + xt * xt
                c_r = cols[r]
                off = pl.multiple_of(
                    jnp.minimum(c_r & ~(_L - 1), t - (t % _L)), _L
                )
                xg = buf_v[par, r, pl.ds(off, _L)]
                g = g + jnp.where(lanes == c_r - off, xg, 0.0)
            return acc, g

        def body(i, carry):
            accs2, accg = carry
            k0 = 2 * i
            accs2, accg = consume(0, k0, accs2, accg)
            pltpu.async_copy(
                x_hbm.at[pl.ds(base + (k0 + 2) * _R, _R)], buf_v.at[0], sem0
            )
            accs2, accg = consume(1, k0 + 1, accs2, accg)
            pltpu.async_copy(
                x_hbm.at[pl.ds(base + (k0 + 3) * _R, _R)], buf_v.at[1], sem1
            )
            return accs2, accg

        zero = jnp.zeros((_L,), jnp.float32)
        accs2, accg = lax.fori_loop(0, nchunk // 2 - 1, body, (zero, zero))
        accs2, accg = consume(0, nchunk - 2, accs2, accg)
        accs2, accg = consume(1, nchunk - 1, accs2, accg)

        acc_v[...] = accs2 - 2.0 * accg
        pltpu.sync_copy(acc_v, out_hbm.at[wid])

    return sc_loss


def kernel(lambda_t, Lambda_t, c):
    b, t = Lambda_t.shape
    partials = _make_sc_loss(b, t)(Lambda_t, c.reshape(-1))
    return (jnp.sum(partials) + jnp.float32(b)) / jnp.float32(b * t)


# speedup vs baseline: 1.8215x; 1.8215x over previous
import functools

import jax
import jax.numpy as jnp
from jax import lax
from jax.experimental import pallas as pl
from jax.experimental.pallas import tpu as pltpu
from jax.experimental.pallas import tpu_sc as plsc

_NC = 2
_NS = 16
_L = 16
_R = 16


@functools.cache
def _make_sc_loss(b, t):
    nw = _NC * _NS
    bw = b // nw
    nchunk = bw // _R
    nfull = t // _L
    assert nchunk % 2 == 0 and nchunk >= 4 and t % _L != 0
    mesh = plsc.VectorSubcoreMesh(core_axis_name="c", subcore_axis_name="s")

    @functools.partial(
        pl.kernel,
        mesh=mesh,
        out_type=jax.ShapeDtypeStruct((nw, _L), jnp.float32),
        scratch_types=[
            pltpu.VMEM((bw,), jnp.int32),
            pltpu.VMEM((2, _R, t), jnp.float32),
            pltpu.VMEM((_L,), jnp.float32),
            pltpu.SemaphoreType.DMA,
            pltpu.SemaphoreType.DMA,
        ],
    )
    def sc_loss(x_hbm, c_hbm, out_hbm, c_v, buf_v, acc_v, sem0, sem1):
        wid = lax.axis_index("s") * _NC + lax.axis_index("c")
        base = wid * bw
        pltpu.sync_copy(c_hbm.at[pl.ds(base, bw)], c_v)

        lanes = lax.iota(jnp.int32, _L)
        tailmask = jnp.where(lanes >= _L - (t % _L), 1.0, 0.0).astype(jnp.float32)
        sems = (sem0, sem1)

        def consume(par, k, accs, accg):
            pltpu.make_async_copy(
                x_hbm.at[pl.ds(0, _R)], buf_v.at[par], sems[par]
            ).wait()

            def inner(o, acc_t):
                acc_l = list(acc_t)
                off = pl.multiple_of(o * _L, _L)
                for r in range(_R):
                    x = buf_v[par, r, pl.ds(off, _L)]
                    acc_l[r & 7] = acc_l[r & 7] + x * x
                return tuple(acc_l)

            accs = lax.fori_loop(0, nfull, inner, tuple(accs))

            g = accg
            cols = c_v[pl.ds(pl.multiple_of(k * _R, _L), _L)]
            accs = list(accs)
            for r in range(_R):
                xt = buf_v[par, r, pl.ds(t - _L, _L)] * tailmask
                accs[r & 7] = accs[r & 7] + xt * xt
                c_r = cols[r]
                off = pl.multiple_of(
                    jnp.minimum(c_r & ~(_L - 1), t - (t % _L)), _L
                )
                xg = buf_v[par, r, pl.ds(off, _L)]
                g = g + jnp.where(lanes == c_r - off, xg, 0.0)
            return tuple(accs), g

        pltpu.async_copy(x_hbm.at[pl.ds(base, _R)], buf_v.at[0], sem0)
        pltpu.async_copy(x_hbm.at[pl.ds(base + _R, _R)], buf_v.at[1], sem1)

        def body(i, carry):
            accs, accg = carry
            k0 = 2 * i
            accs, accg = consume(0, k0, accs, accg)
            pltpu.async_copy(
                x_hbm.at[pl.ds(base + (k0 + 2) * _R, _R)], buf_v.at[0], sem0
            )
            accs, accg = consume(1, k0 + 1, accs, accg)
            pltpu.async_copy(
                x_hbm.at[pl.ds(base + (k0 + 3) * _R, _R)], buf_v.at[1], sem1
            )
            return accs, accg

        zero = jnp.zeros((_L,), jnp.float32)
        accs, accg = lax.fori_loop(
            0, nchunk // 2 - 1, body, ((zero,) * 8, zero)
        )
        accs, accg = consume(0, nchunk - 2, accs, accg)
        accs, accg = consume(1, nchunk - 1, accs, accg)

        accs2 = (
            ((accs[0] + accs[1]) + (accs[2] + accs[3]))
            + ((accs[4] + accs[5]) + (accs[6] + accs[7]))
        )
        acc_v[...] = accs2 - 2.0 * accg
        pltpu.sync_copy(acc_v, out_hbm.at[wid])

    return sc_loss


def kernel(lambda_t, Lambda_t, c):
    b, t = Lambda_t.shape
    partials = _make_sc_loss(b, t)(Lambda_t, c.reshape(-1))
    return (jnp.sum(partials) + jnp.float32(b)) / jnp.float32(b * t)


# speedup vs baseline: 1.9686x; 1.0807x over previous
import functools

import jax
import jax.numpy as jnp
from jax import lax
from jax.experimental import pallas as pl
from jax.experimental.pallas import tpu as pltpu
from jax.experimental.pallas import tpu_sc as plsc

_NC = 2
_NS = 16
_L = 16
_R = 16
_NBUF = 4


@functools.cache
def _make_sc_loss(b, t):
    nw = _NC * _NS
    bw = b // nw
    nchunk = bw // _R
    nfull = t // _L
    assert nchunk % _NBUF == 0 and nchunk >= 2 * _NBUF and t % _L != 0
    mesh = plsc.VectorSubcoreMesh(core_axis_name="c", subcore_axis_name="s")

    @functools.partial(
        pl.kernel,
        mesh=mesh,
        out_type=jax.ShapeDtypeStruct((nw, _L), jnp.float32),
        scratch_types=[
            pltpu.VMEM((bw,), jnp.int32),
            pltpu.VMEM((_NBUF, _R, t), jnp.float32),
            pltpu.VMEM((_L,), jnp.float32),
            pltpu.SemaphoreType.DMA,
            pltpu.SemaphoreType.DMA,
            pltpu.SemaphoreType.DMA,
            pltpu.SemaphoreType.DMA,
        ],
    )
    def sc_loss(
        x_hbm, c_hbm, out_hbm, c_v, buf_v, acc_v, sem0, sem1, sem2, sem3
    ):
        wid = lax.axis_index("s") * _NC + lax.axis_index("c")
        base = wid * bw
        pltpu.sync_copy(c_hbm.at[pl.ds(base, bw)], c_v)

        lanes = lax.iota(jnp.int32, _L)
        tailmask = jnp.where(lanes >= _L - (t % _L), 1.0, 0.0).astype(jnp.float32)
        sems = (sem0, sem1, sem2, sem3)

        def consume(par, k, accs, accg):
            pltpu.make_async_copy(
                x_hbm.at[pl.ds(0, _R)], buf_v.at[par], sems[par]
            ).wait()

            def inner(o, acc_t):
                acc_l = list(acc_t)
                off = pl.multiple_of(o * _L, _L)
                for r in range(_R):
                    x = buf_v[par, r, pl.ds(off, _L)]
                    acc_l[r & 7] = acc_l[r & 7] + x * x
                return tuple(acc_l)

            accs = lax.fori_loop(0, nfull, inner, tuple(accs))

            g = accg
            cols = c_v[pl.ds(pl.multiple_of(k * _R, _L), _L)]
            accs = list(accs)
            for r in range(_R):
                xt = buf_v[par, r, pl.ds(t - _L, _L)] * tailmask
                accs[r & 7] = accs[r & 7] + xt * xt
                c_r = cols[r]
                off = pl.multiple_of(
                    jnp.minimum(c_r & ~(_L - 1), t - (t % _L)), _L
                )
                xg = buf_v[par, r, pl.ds(off, _L)]
                g = g + jnp.where(lanes == c_r - off, xg, 0.0)
            return tuple(accs), g

        def issue(par, k):
            pltpu.async_copy(
                x_hbm.at[pl.ds(base + k * _R, _R)], buf_v.at[par], sems[par]
            )

        for par in range(_NBUF):
            issue(par, par)

        def body(i, carry):
            accs, accg = carry
            k0 = _NBUF * i
            for par in range(_NBUF):
                accs, accg = consume(par, k0 + par, accs, accg)
                issue(par, k0 + par + _NBUF)
            return accs, accg

        zero = jnp.zeros((_L,), jnp.float32)
        accs, accg = lax.fori_loop(
            0, nchunk // _NBUF - 1, body, ((zero,) * 8, zero)
        )
        for par in range(_NBUF):
            accs, accg = consume(par, nchunk - _NBUF + par, accs, accg)

        accs2 = (
            ((accs[0] + accs[1]) + (accs[2] + accs[3]))
            + ((accs[4] + accs[5]) + (accs[6] + accs[7]))
        )
        acc_v[...] = accs2 - 2.0 * accg
        pltpu.sync_copy(acc_v, out_hbm.at[wid])

    return sc_loss


def kernel(lambda_t, Lambda_t, c):
    b, t = Lambda_t.shape
    partials = _make_sc_loss(b, t)(Lambda_t, c.reshape(-1))
    return (jnp.sum(partials) + jnp.float32(b)) / jnp.float32(b * t)


# speedup vs baseline: 2.0183x; 1.0253x over previous
import functools

import jax
import jax.numpy as jnp
from jax import lax
from jax.experimental import pallas as pl
from jax.experimental.pallas import tpu as pltpu
from jax.experimental.pallas import tpu_sc as plsc

_NC = 2
_NS = 16
_L = 16
_R = 16
_NBUF = 4

_SPLIT = 8192
_TC_BLOCK = 1024


def _tc_body(x_ref, c_ref, out_ref):
    @pl.when(pl.program_id(0) == 0)
    def _init():
        out_ref[0, 0] = jnp.float32(0.0)
        out_ref[0, 1] = jnp.float32(0.0)

    x = x_ref[...]
    cols = lax.broadcasted_iota(jnp.int32, x.shape, 1)
    out_ref[0, 0] += jnp.sum(x * x)
    out_ref[0, 1] += jnp.sum(jnp.where(cols == c_ref[...][:, None], x, 0.0))


@functools.partial(jax.jit, static_argnums=(2, 3))
def _tc_part(x, c, rows, block_rows):
    t = x.shape[1]
    return pl.pallas_call(
        _tc_body,
        grid=(rows // block_rows,),
        in_specs=[
            pl.BlockSpec((block_rows, t), lambda i: (i, 0)),
            pl.BlockSpec((block_rows,), lambda i: (i,)),
        ],
        out_specs=pl.BlockSpec(memory_space=pltpu.SMEM),
        out_shape=jax.ShapeDtypeStruct((1, 2), jnp.float32),
    )(x, c)


@functools.cache
def _make_sc_loss(b, t, start):
    nw = _NC * _NS
    bw = (b - start) // nw
    nchunk = bw // _R
    nfull = t // _L
    assert nchunk % _NBUF == 0 and nchunk >= 2 * _NBUF and t % _L != 0
    mesh = plsc.VectorSubcoreMesh(core_axis_name="c", subcore_axis_name="s")

    @functools.partial(
        pl.kernel,
        mesh=mesh,
        out_type=jax.ShapeDtypeStruct((nw, _L), jnp.float32),
        scratch_types=[
            pltpu.VMEM((bw,), jnp.int32),
            pltpu.VMEM((_NBUF, _R, t), jnp.float32),
            pltpu.VMEM((_L,), jnp.float32),
            pltpu.SemaphoreType.DMA,
            pltpu.SemaphoreType.DMA,
            pltpu.SemaphoreType.DMA,
            pltpu.SemaphoreType.DMA,
        ],
    )
    def sc_loss(
        x_hbm, c_hbm, out_hbm, c_v, buf_v, acc_v, sem0, sem1, sem2, sem3
    ):
        wid = lax.axis_index("s") * _NC + lax.axis_index("c")
        base = start + wid * bw
        pltpu.sync_copy(c_hbm.at[pl.ds(base, bw)], c_v)

        lanes = lax.iota(jnp.int32, _L)
        tailmask = jnp.where(lanes >= _L - (t % _L), 1.0, 0.0).astype(jnp.float32)
        sems = (sem0, sem1, sem2, sem3)

        def consume(par, k, accs, accg):
            pltpu.make_async_copy(
                x_hbm.at[pl.ds(0, _R)], buf_v.at[par], sems[par]
            ).wait()

            def inner(o, acc_t):
                acc_l = list(acc_t)
                off = pl.multiple_of(o * _L, _L)
                for r in range(_R):
                    x = buf_v[par, r, pl.ds(off, _L)]
                    acc_l[r & 7] = acc_l[r & 7] + x * x
                return tuple(acc_l)

            accs = lax.fori_loop(0, nfull, inner, tuple(accs))

            g = accg
            cols = c_v[pl.ds(pl.multiple_of(k * _R, _L), _L)]
            accs = list(accs)
            for r in range(_R):
                xt = buf_v[par, r, pl.ds(t - _L, _L)] * tailmask
                accs[r & 7] = accs[r & 7] + xt * xt
                c_r = cols[r]
                off = pl.multiple_of(
                    jnp.minimum(c_r & ~(_L - 1), t - (t % _L)), _L
                )
                xg = buf_v[par, r, pl.ds(off, _L)]
                g = g + jnp.where(lanes == c_r - off, xg, 0.0)
            return tuple(accs), g

        def issue(par, k):
            pltpu.async_copy(
                x_hbm.at[pl.ds(base + k * _R, _R)], buf_v.at[par], sems[par]
            )

        for par in range(_NBUF):
            issue(par, par)

        def body(i, carry):
            accs, accg = carry
            k0 = _NBUF * i
            for par in range(_NBUF):
                accs, accg = consume(par, k0 + par, accs, accg)
                issue(par, k0 + par + _NBUF)
            return accs, accg

        zero = jnp.zeros((_L,), jnp.float32)
        accs, accg = lax.fori_loop(
            0, nchunk // _NBUF - 1, body, ((zero,) * 8, zero)
        )
        for par in range(_NBUF):
            accs, accg = consume(par, nchunk - _NBUF + par, accs, accg)

        accs2 = (
            ((accs[0] + accs[1]) + (accs[2] + accs[3]))
            + ((accs[4] + accs[5]) + (accs[6] + accs[7]))
        )
        acc_v[...] = accs2 - 2.0 * accg
        pltpu.sync_copy(acc_v, out_hbm.at[wid])

    return sc_loss


def kernel(lambda_t, Lambda_t, c):
    b, t = Lambda_t.shape
    c_flat = c.reshape(-1)
    partials = _make_sc_loss(b, t, _SPLIT)(Lambda_t, c_flat)
    tc_out = _tc_part(Lambda_t, c_flat, _SPLIT, _TC_BLOCK)
    total = jnp.sum(partials) + tc_out[0, 0] - 2.0 * tc_out[0, 1]
    return (total + jnp.float32(b)) / jnp.float32(b * t)


# speedup vs baseline: 2.0228x; 1.0022x over previous
import functools

import jax
import jax.numpy as jnp
from jax import lax
from jax.experimental import pallas as pl
from jax.experimental.pallas import tpu as pltpu
from jax.experimental.pallas import tpu_sc as plsc

_NC = 2
_NS = 16
_L = 16
_R = 16
_NBUF = 4

_SPLIT = 6144
_TC_BLOCK = 1024


def _tc_body(x_ref, c_ref, out_ref):
    @pl.when(pl.program_id(0) == 0)
    def _init():
        out_ref[0, 0] = jnp.float32(0.0)
        out_ref[0, 1] = jnp.float32(0.0)

    x = x_ref[...]
    cols = lax.broadcasted_iota(jnp.int32, x.shape, 1)
    out_ref[0, 0] += jnp.sum(x * x)
    out_ref[0, 1] += jnp.sum(jnp.where(cols == c_ref[...][:, None], x, 0.0))


@functools.partial(jax.jit, static_argnums=(2, 3))
def _tc_part(x, c, rows, block_rows):
    t = x.shape[1]
    return pl.pallas_call(
        _tc_body,
        grid=(rows // block_rows,),
        in_specs=[
            pl.BlockSpec((block_rows, t), lambda i: (i, 0)),
            pl.BlockSpec((block_rows,), lambda i: (i,)),
        ],
        out_specs=pl.BlockSpec(memory_space=pltpu.SMEM),
        out_shape=jax.ShapeDtypeStruct((1, 2), jnp.float32),
    )(x, c)


@functools.cache
def _make_sc_loss(b, t, start):
    nw = _NC * _NS
    bw = (b - start) // nw
    nchunk = bw // _R
    nfull = t // _L
    assert nchunk % _NBUF == 0 and nchunk >= 2 * _NBUF and t % _L != 0
    mesh = plsc.VectorSubcoreMesh(core_axis_name="c", subcore_axis_name="s")

    @functools.partial(
        pl.kernel,
        mesh=mesh,
        out_type=jax.ShapeDtypeStruct((nw, _L), jnp.float32),
        scratch_types=[
            pltpu.VMEM((bw,), jnp.int32),
            pltpu.VMEM((_NBUF, _R, t), jnp.float32),
            pltpu.VMEM((_L,), jnp.float32),
            pltpu.SemaphoreType.DMA,
            pltpu.SemaphoreType.DMA,
            pltpu.SemaphoreType.DMA,
            pltpu.SemaphoreType.DMA,
        ],
    )
    def sc_loss(
        x_hbm, c_hbm, out_hbm, c_v, buf_v, acc_v, sem0, sem1, sem2, sem3
    ):
        wid = lax.axis_index("s") * _NC + lax.axis_index("c")
        base = start + wid * bw
        pltpu.sync_copy(c_hbm.at[pl.ds(base, bw)], c_v)

        lanes = lax.iota(jnp.int32, _L)
        tailmask = jnp.where(lanes >= _L - (t % _L), 1.0, 0.0).astype(jnp.float32)
        sems = (sem0, sem1, sem2, sem3)

        def consume(par, k, accs, accg):
            pltpu.make_async_copy(
                x_hbm.at[pl.ds(0, _R)], buf_v.at[par], sems[par]
            ).wait()

            def inner(o, acc_t):
                acc_l = list(acc_t)
                off = pl.multiple_of(o * _L, _L)
                for r in range(_R):
                    x = buf_v[par, r, pl.ds(off, _L)]
                    acc_l[r & 7] = acc_l[r & 7] + x * x
                return tuple(acc_l)

            accs = lax.fori_loop(0, nfull, inner, tuple(accs))

            g = accg
            cols = c_v[pl.ds(pl.multiple_of(k * _R, _L), _L)]
            accs = list(accs)
            for r in range(_R):
                xt = buf_v[par, r, pl.ds(t - _L, _L)] * tailmask
                accs[r & 7] = accs[r & 7] + xt * xt
                c_r = cols[r]
                off = pl.multiple_of(
                    jnp.minimum(c_r & ~(_L - 1), t - (t % _L)), _L
                )
                xg = buf_v[par, r, pl.ds(off, _L)]
                g = g + jnp.where(lanes == c_r - off, xg, 0.0)
            return tuple(accs), g

        def issue(par, k):
            pltpu.async_copy(
                x_hbm.at[pl.ds(base + k * _R, _R)], buf_v.at[par], sems[par]
            )

        for par in range(_NBUF):
            issue(par, par)

        def body(i, carry):
            accs, accg = carry
            k0 = _NBUF * i
            for par in range(_NBUF):
                accs, accg = consume(par, k0 + par, accs, accg)
                issue(par, k0 + par + _NBUF)
            return accs, accg

        zero = jnp.zeros((_L,), jnp.float32)
        accs, accg = lax.fori_loop(
            0, nchunk // _NBUF - 1, body, ((zero,) * 8, zero)
        )
        for par in range(_NBUF):
            accs, accg = consume(par, nchunk - _NBUF + par, accs, accg)

        accs2 = (
            ((accs[0] + accs[1]) + (accs[2] + accs[3]))
            + ((accs[4] + accs[5]) + (accs[6] + accs[7]))
        )
        acc_v[...] = accs2 - 2.0 * accg
        pltpu.sync_copy(acc_v, out_hbm.at[wid])

    return sc_loss


def kernel(lambda_t, Lambda_t, c):
    b, t = Lambda_t.shape
    c_flat = c.reshape(-1)
    partials = _make_sc_loss(b, t, _SPLIT)(Lambda_t, c_flat)
    tc_out = _tc_part(Lambda_t, c_flat, _SPLIT, _TC_BLOCK)
    total = jnp.sum(partials) + tc_out[0, 0] - 2.0 * tc_out[0, 1]
    return (total + jnp.float32(b)) / jnp.float32(b * t)


# speedup vs baseline: 2.0464x; 1.0117x over previous
import functools

import jax
import jax.numpy as jnp
from jax import lax
from jax.experimental import pallas as pl
from jax.experimental.pallas import tpu as pltpu
from jax.experimental.pallas import tpu_sc as plsc

_NC = 2
_NS = 16
_L = 16
_R = 16
_NBUF = 4

_SPLIT = 6144
_TC_BLOCK = 2048


def _tc_body(x_ref, c_ref, out_ref):
    @pl.when(pl.program_id(0) == 0)
    def _init():
        out_ref[0, 0] = jnp.float32(0.0)
        out_ref[0, 1] = jnp.float32(0.0)

    x = x_ref[...]
    cols = lax.broadcasted_iota(jnp.int32, x.shape, 1)
    out_ref[0, 0] += jnp.sum(x * x)
    out_ref[0, 1] += jnp.sum(jnp.where(cols == c_ref[...][:, None], x, 0.0))


@functools.partial(jax.jit, static_argnums=(2, 3))
def _tc_part(x, c, rows, block_rows):
    t = x.shape[1]
    return pl.pallas_call(
        _tc_body,
        grid=(rows // block_rows,),
        in_specs=[
            pl.BlockSpec((block_rows, t), lambda i: (i, 0)),
            pl.BlockSpec((block_rows,), lambda i: (i,)),
        ],
        out_specs=pl.BlockSpec(memory_space=pltpu.SMEM),
        out_shape=jax.ShapeDtypeStruct((1, 2), jnp.float32),
    )(x, c)


@functools.cache
def _make_sc_loss(b, t, start):
    nw = _NC * _NS
    bw = (b - start) // nw
    nchunk = bw // _R
    nfull = t // _L
    assert nchunk % _NBUF == 0 and nchunk >= 2 * _NBUF and t % _L != 0
    mesh = plsc.VectorSubcoreMesh(core_axis_name="c", subcore_axis_name="s")

    @functools.partial(
        pl.kernel,
        mesh=mesh,
        out_type=jax.ShapeDtypeStruct((nw, _L), jnp.float32),
        scratch_types=[
            pltpu.VMEM((bw,), jnp.int32),
            pltpu.VMEM((_NBUF, _R, t), jnp.float32),
            pltpu.VMEM((_L,), jnp.float32),
            pltpu.SemaphoreType.DMA,
            pltpu.SemaphoreType.DMA,
            pltpu.SemaphoreType.DMA,
            pltpu.SemaphoreType.DMA,
        ],
    )
    def sc_loss(
        x_hbm, c_hbm, out_hbm, c_v, buf_v, acc_v, sem0, sem1, sem2, sem3
    ):
        wid = lax.axis_index("s") * _NC + lax.axis_index("c")
        base = start + wid * bw
        pltpu.sync_copy(c_hbm.at[pl.ds(base, bw)], c_v)

        lanes = lax.iota(jnp.int32, _L)
        tailmask = jnp.where(lanes >= _L - (t % _L), 1.0, 0.0).astype(jnp.float32)
        sems = (sem0, sem1, sem2, sem3)

        def consume(par, k, accs, accg):
            pltpu.make_async_copy(
                x_hbm.at[pl.ds(0, _R)], buf_v.at[par], sems[par]
            ).wait()

            def inner(o, acc_t):
                acc_l = list(acc_t)
                off = pl.multiple_of(o * _L, _L)
                for r in range(_R):
                    x = buf_v[par, r, pl.ds(off, _L)]
                    acc_l[r & 7] = acc_l[r & 7] + x * x
                return tuple(acc_l)

            accs = lax.fori_loop(0, nfull, inner, tuple(accs))

            g = accg
            cols = c_v[pl.ds(pl.multiple_of(k * _R, _L), _L)]
            accs = list(accs)
            for r in range(_R):
                xt = buf_v[par, r, pl.ds(t - _L, _L)] * tailmask
                accs[r & 7] = accs[r & 7] + xt * xt
                c_r = cols[r]
                off = pl.multiple_of(
                    jnp.minimum(c_r & ~(_L - 1), t - (t % _L)), _L
                )
                xg = buf_v[par, r, pl.ds(off, _L)]
                g = g + jnp.where(lanes == c_r - off, xg, 0.0)
            return tuple(accs), g

        def issue(par, k):
            pltpu.async_copy(
                x_hbm.at[pl.ds(base + k * _R, _R)], buf_v.at[par], sems[par]
            )

        for par in range(_NBUF):
            issue(par, par)

        def body(i, carry):
            accs, accg = carry
            k0 = _NBUF * i
            for par in range(_NBUF):
                accs, accg = consume(par, k0 + par, accs, accg)
                issue(par, k0 + par + _NBUF)
            return accs, accg

        zero = jnp.zeros((_L,), jnp.float32)
        accs, accg = lax.fori_loop(
            0, nchunk // _NBUF - 1, body, ((zero,) * 8, zero)
        )
        for par in range(_NBUF):
            accs, accg = consume(par, nchunk - _NBUF + par, accs, accg)

        accs2 = (
            ((accs[0] + accs[1]) + (accs[2] + accs[3]))
            + ((accs[4] + accs[5]) + (accs[6] + accs[7]))
        )
        acc_v[...] = accs2 - 2.0 * accg
        pltpu.sync_copy(acc_v, out_hbm.at[wid])

    return sc_loss


def kernel(lambda_t, Lambda_t, c):
    b, t = Lambda_t.shape
    c_flat = c.reshape(-1)
    partials = _make_sc_loss(b, t, _SPLIT)(Lambda_t, c_flat)
    tc_out = _tc_part(Lambda_t, c_flat, _SPLIT, _TC_BLOCK)
    total = jnp.sum(partials) + tc_out[0, 0] - 2.0 * tc_out[0, 1]
    return (total + jnp.float32(b)) / jnp.float32(b * t)


# speedup vs baseline: 4.5589x; 2.2278x over previous
import functools

import jax
import jax.numpy as jnp
from jax import lax
from jax.experimental import pallas as pl
from jax.experimental.pallas import tpu as pltpu
from jax.experimental.pallas import tpu_sc as plsc

_NC = 2
_NS = 16
_L = 16
_CR = 8
_NBUF = 4

_SPLIT = 680
_TC_BLOCK = 2048


def _tc_body(x_ref, c_ref, out_ref):
    @pl.when(pl.program_id(0) == 0)
    def _init():
        out_ref[0, 0] = jnp.float32(0.0)
        out_ref[0, 1] = jnp.float32(0.0)

    x = x_ref[...]
    rio = lax.broadcasted_iota(jnp.int32, x.shape, 0)
    out_ref[0, 0] += jnp.sum(x * x)
    out_ref[0, 1] += jnp.sum(jnp.where(rio == c_ref[...][None, :], x, 0.0))


@functools.partial(jax.jit, static_argnums=(2, 3))
def _tc_part(xt, c, rows, block_cols):
    n = xt.shape[1]
    return pl.pallas_call(
        _tc_body,
        grid=(n // block_cols,),
        in_specs=[
            pl.BlockSpec((rows, block_cols), lambda j: (0, j)),
            pl.BlockSpec((block_cols,), lambda j: (j,)),
        ],
        out_specs=pl.BlockSpec(memory_space=pltpu.SMEM),
        out_shape=jax.ShapeDtypeStruct((1, 2), jnp.float32),
    )(xt, c)


@functools.cache
def _make_sc_loss(t, n, start):
    nw = _NC * _NS
    cw = n // nw
    nchunk = (t - start) // _CR
    nwin = cw // _L
    assert nchunk % _NBUF == 0 and nchunk >= 2 * _NBUF
    assert cw % _L == 0 and start % _CR == 0
    mesh = plsc.VectorSubcoreMesh(core_axis_name="c", subcore_axis_name="s")

    @functools.partial(
        pl.kernel,
        mesh=mesh,
        out_type=jax.ShapeDtypeStruct((nw, _L), jnp.float32),
        scratch_types=[
            pltpu.VMEM((cw,), jnp.int32),
            pltpu.VMEM((_NBUF, _CR, cw), jnp.float32),
            pltpu.VMEM((_L,), jnp.float32),
            pltpu.SemaphoreType.DMA,
            pltpu.SemaphoreType.DMA,
            pltpu.SemaphoreType.DMA,
            pltpu.SemaphoreType.DMA,
        ],
    )
    def sc_loss(
        xt_hbm, c_hbm, out_hbm, c_v, buf_v, acc_v, sem0, sem1, sem2, sem3
    ):
        wid = lax.axis_index("s") * _NC + lax.axis_index("c")
        cbase = wid * cw
        pltpu.sync_copy(c_hbm.at[pl.ds(cbase, cw)], c_v)
        sems = (sem0, sem1, sem2, sem3)

        def consume(par, k, accs, accg):
            pltpu.make_async_copy(
                xt_hbm.at[pl.ds(0, _CR), pl.ds(0, cw)], buf_v.at[par],
                sems[par],
            ).wait()
            j0 = start + k * _CR

            def inner(w, carry):
                acc_l = list(carry[0])
                g_l = list(carry[1])
                off = pl.multiple_of(w * _L, _L)
                cv = c_v[pl.ds(off, _L)]
                for r in range(_CR):
                    x = buf_v[par, r, pl.ds(off, _L)]
                    acc_l[r] = acc_l[r] + x * x
                    g_l[r & 3] = g_l[r & 3] + jnp.where(
                        cv == j0 + r, x, 0.0
                    )
                return tuple(acc_l), tuple(g_l)

            return lax.fori_loop(0, nwin, inner, (accs, accg))

        def issue(par, k):
            pltpu.async_copy(
                xt_hbm.at[pl.ds(start + k * _CR, _CR), pl.ds(cbase, cw)],
                buf_v.at[par],
                sems[par],
            )

        for par in range(_NBUF):
            issue(par, par)

        def body(i, carry):
            accs, accg = carry
            k0 = _NBUF * i
            for par in range(_NBUF):
                accs, accg = consume(par, k0 + par, accs, accg)
                issue(par, k0 + par + _NBUF)
            return accs, accg

        zero = jnp.zeros((_L,), jnp.float32)
        accs, accg = lax.fori_loop(
            0, nchunk // _NBUF - 1, body, ((zero,) * _CR, (zero,) * 4)
        )
        for par in range(_NBUF):
            accs, accg = consume(par, nchunk - _NBUF + par, accs, accg)

        accs2 = (
            ((accs[0] + accs[1]) + (accs[2] + accs[3]))
            + ((accs[4] + accs[5]) + (accs[6] + accs[7]))
        )
        gsum = (accg[0] + accg[1]) + (accg[2] + accg[3])
        acc_v[...] = accs2 - 2.0 * gsum
        pltpu.sync_copy(acc_v, out_hbm.at[wid])

    return sc_loss


def kernel(lambda_t, Lambda_t, c):
    b, t = Lambda_t.shape
    xt = Lambda_t.T
    c_flat = c.reshape(-1)
    partials = _make_sc_loss(t, b, _SPLIT)(xt, c_flat)
    tc_out = _tc_part(xt, c_flat, _SPLIT, _TC_BLOCK)
    total = jnp.sum(partials) + tc_out[0, 0] - 2.0 * tc_out[0, 1]
    return (total + jnp.float32(b)) / jnp.float32(b * t)


# speedup vs baseline: 4.6353x; 1.0167x over previous
import functools

import jax
import jax.numpy as jnp
from jax import lax
from jax.experimental import pallas as pl
from jax.experimental.pallas import tpu as pltpu
from jax.experimental.pallas import tpu_sc as plsc

_NC = 2
_NS = 16
_L = 16
_CR = 8
_NBUF = 4

_SPLIT = 616
_TC_BLOCK = 2048


def _tc_body(x_ref, c_ref, out_ref):
    @pl.when(pl.program_id(0) == 0)
    def _init():
        out_ref[0, 0] = jnp.float32(0.0)
        out_ref[0, 1] = jnp.float32(0.0)

    x = x_ref[...]
    rio = lax.broadcasted_iota(jnp.int32, x.shape, 0)
    out_ref[0, 0] += jnp.sum(x * x)
    out_ref[0, 1] += jnp.sum(jnp.where(rio == c_ref[...][None, :], x, 0.0))


@functools.partial(jax.jit, static_argnums=(2, 3))
def _tc_part(xt, c, rows, block_cols):
    n = xt.shape[1]
    return pl.pallas_call(
        _tc_body,
        grid=(n // block_cols,),
        in_specs=[
            pl.BlockSpec((rows, block_cols), lambda j: (0, j)),
            pl.BlockSpec((block_cols,), lambda j: (j,)),
        ],
        out_specs=pl.BlockSpec(memory_space=pltpu.SMEM),
        out_shape=jax.ShapeDtypeStruct((1, 2), jnp.float32),
    )(xt, c)


@functools.cache
def _make_sc_loss(t, n, start):
    nw = _NC * _NS
    cw = n // nw
    nchunk = (t - start) // _CR
    nwin = cw // _L
    assert nchunk % _NBUF == 0 and nchunk >= 2 * _NBUF
    assert cw % _L == 0 and start % _CR == 0
    mesh = plsc.VectorSubcoreMesh(core_axis_name="c", subcore_axis_name="s")

    @functools.partial(
        pl.kernel,
        mesh=mesh,
        out_type=jax.ShapeDtypeStruct((nw, _L), jnp.float32),
        scratch_types=[
            pltpu.VMEM((cw,), jnp.int32),
            pltpu.VMEM((_NBUF, _CR, cw), jnp.float32),
            pltpu.VMEM((_L,), jnp.float32),
            pltpu.SemaphoreType.DMA,
            pltpu.SemaphoreType.DMA,
            pltpu.SemaphoreType.DMA,
            pltpu.SemaphoreType.DMA,
        ],
    )
    def sc_loss(
        xt_hbm, c_hbm, out_hbm, c_v, buf_v, acc_v, sem0, sem1, sem2, sem3
    ):
        wid = lax.axis_index("s") * _NC + lax.axis_index("c")
        cbase = wid * cw
        pltpu.sync_copy(c_hbm.at[pl.ds(cbase, cw)], c_v)
        sems = (sem0, sem1, sem2, sem3)

        def consume(par, k, accs, accg):
            pltpu.make_async_copy(
                xt_hbm.at[pl.ds(0, _CR), pl.ds(0, cw)], buf_v.at[par],
                sems[par],
            ).wait()
            j0 = start + k * _CR

            def inner(w, carry):
                acc_l = list(carry[0])
                g_l = list(carry[1])
                off = pl.multiple_of(w * _L, _L)
                cv = c_v[pl.ds(off, _L)]
                for r in range(_CR):
                    x = buf_v[par, r, pl.ds(off, _L)]
                    acc_l[r] = acc_l[r] + x * x
                    g_l[r & 3] = g_l[r & 3] + jnp.where(
                        cv == j0 + r, x, 0.0
                    )
                return tuple(acc_l), tuple(g_l)

            return lax.fori_loop(0, nwin, inner, (accs, accg))

        def issue(par, k):
            pltpu.async_copy(
                xt_hbm.at[pl.ds(start + k * _CR, _CR), pl.ds(cbase, cw)],
                buf_v.at[par],
                sems[par],
            )

        for par in range(_NBUF):
            issue(par, par)

        def body(i, carry):
            accs, accg = carry
            k0 = _NBUF * i
            for par in range(_NBUF):
                accs, accg = consume(par, k0 + par, accs, accg)
                issue(par, k0 + par + _NBUF)
            return accs, accg

        zero = jnp.zeros((_L,), jnp.float32)
        accs, accg = lax.fori_loop(
            0, nchunk // _NBUF - 1, body, ((zero,) * _CR, (zero,) * 4)
        )
        for par in range(_NBUF):
            accs, accg = consume(par, nchunk - _NBUF + par, accs, accg)

        accs2 = (
            ((accs[0] + accs[1]) + (accs[2] + accs[3]))
            + ((accs[4] + accs[5]) + (accs[6] + accs[7]))
        )
        gsum = (accg[0] + accg[1]) + (accg[2] + accg[3])
        acc_v[...] = accs2 - 2.0 * gsum
        pltpu.sync_copy(acc_v, out_hbm.at[wid])

    return sc_loss


def kernel(lambda_t, Lambda_t, c):
    b, t = Lambda_t.shape
    xt = Lambda_t.T
    c_flat = c.reshape(-1)
    partials = _make_sc_loss(t, b, _SPLIT)(xt, c_flat)
    tc_out = _tc_part(xt, c_flat, _SPLIT, _TC_BLOCK)
    total = jnp.sum(partials) + tc_out[0, 0] - 2.0 * tc_out[0, 1]
    return (total + jnp.float32(b)) / jnp.float32(b * t)
